# baseline (speedup 1.0000x reference)
import functools

import jax
import jax.numpy as jnp
from jax import lax
from jax.experimental import pallas as pl
from jax.experimental.pallas import tpu as pltpu
from jax.experimental.pallas import tpu_sc as plsc

VOCAB = 1000000
EMB = 64
NUM_CLASS = 1000
B = 4096
L = 200

NC = 2
NS = 16
NW = NC * NS
BPW = B // NW
CHUNKS = ((0, 104), (104, 96))


def _pool_body(idx_hbm, table_hbm, out_hbm, idx_v, rows0, rows1, out_v,
               sem0, sem1):
    c = lax.axis_index("c")
    s = lax.axis_index("s")
    wid = s * NC + c
    base = wid * BPW

    pltpu.sync_copy(idx_hbm.at[pl.ds(base, BPW)], idx_v)

    def issue(i, buf, sem):
        for off, sz in CHUNKS:
            pltpu.async_copy(
                table_hbm.at[idx_v.at[i, pl.ds(off, sz)]],
                buf.at[pl.ds(off, sz)],
                sem,
            )

    def drain(buf, sem):
        pltpu.make_async_copy(table_hbm.at[pl.ds(0, L)], buf, sem).wait()

    def acc_row(i, buf):
        def acc_body(j4, accs):
            j = j4 * 4
            for u in range(4):
                accs = tuple(accs[k] + buf[j + u, pl.ds(16 * k, 16)]
                             for k in range(4))
            return accs

        accs = (jnp.zeros((16,), jnp.float32),) * 4
        accs = lax.fori_loop(0, L // 4, acc_body, accs)
        for k in range(4):
            out_v[i, pl.ds(16 * k, 16)] = accs[k]

    issue(0, rows0, sem0)

    def pair_body(i2, carry):
        b0 = 2 * i2
        b1 = b0 + 1
        issue(b1, rows1, sem1)
        drain(rows0, sem0)
        acc_row(b0, rows0)
        issue(jnp.minimum(b0 + 2, BPW - 1), rows0, sem0)
        drain(rows1, sem1)
        acc_row(b1, rows1)
        return carry

    lax.fori_loop(0, BPW // 2, pair_body, 0)
    drain(rows0, sem0)
    pltpu.sync_copy(out_v, out_hbm.at[pl.ds(base, BPW)])


def _make_pool():
    mesh = plsc.VectorSubcoreMesh(core_axis_name="c", subcore_axis_name="s")
    return pl.kernel(
        _pool_body,
        out_type=jax.ShapeDtypeStruct((B, EMB), jnp.float32),
        mesh=mesh,
        scratch_types=[
            pltpu.VMEM((BPW, L), jnp.int32),
            pltpu.VMEM((L, EMB), jnp.float32),
            pltpu.VMEM((L, EMB), jnp.float32),
            pltpu.VMEM((BPW, EMB), jnp.float32),
            pltpu.SemaphoreType.DMA,
            pltpu.SemaphoreType.DMA,
        ],
        compiler_params=pltpu.CompilerParams(use_tc_tiling_on_sc=False),
    )


def _mm_body(x_ref, w_ref, b_ref, o_ref):
    x = x_ref[...] * jnp.float32(1.0 / L)
    o_ref[...] = (
        lax.dot_general(w_ref[...], x, (((0,), (1,)), ((), ())),
                        preferred_element_type=jnp.float32)
        + b_ref[...]
    )


def _make_mm(bm):
    return pl.pallas_call(
        _mm_body,
        grid=(B // bm,),
        in_specs=[
            pl.BlockSpec((bm, EMB), lambda i: (i, 0)),
            pl.BlockSpec((EMB, NUM_CLASS), lambda i: (0, 0)),
            pl.BlockSpec((NUM_CLASS, 1), lambda i: (0, 0)),
        ],
        out_specs=pl.BlockSpec((NUM_CLASS, bm), lambda i: (0, i)),
        out_shape=jax.ShapeDtypeStruct((NUM_CLASS, B), jnp.float32),
    )


def kernel(input, table, W, b):
    idx = input.astype(jnp.int32)
    pooled = _make_pool()(idx, table)
    logits_t = _make_mm(512)(pooled, W, b.reshape(NUM_CLASS, 1))
    return logits_t.T


# speedup vs baseline: 1.0286x; 1.0286x over previous
import functools

import jax
import jax.numpy as jnp
from jax import lax
from jax.experimental import pallas as pl
from jax.experimental.pallas import tpu as pltpu
from jax.experimental.pallas import tpu_sc as plsc

VOCAB = 1000000
EMB = 64
NUM_CLASS = 1000
B = 4096
L = 200

NC = 2
NS = 16
NW = NC * NS
BPW = B // NW
CHUNKS = ((0, 104), (104, 96))


def _pool_body(idx_hbm, table_hbm, out_hbm, idx_v, rows0, rows1, out_v,
               sem0, sem1):
    c = lax.axis_index("c")
    s = lax.axis_index("s")
    wid = s * NC + c
    base = wid * BPW

    pltpu.sync_copy(idx_hbm.at[pl.ds(base, BPW)], idx_v)

    def issue(i, buf, sem):
        for off, sz in CHUNKS:
            pltpu.async_copy(
                table_hbm.at[idx_v.at[i, pl.ds(off, sz)]],
                buf.at[pl.ds(off, sz)],
                sem,
            )

    def drain(buf, sem):
        pltpu.make_async_copy(table_hbm.at[pl.ds(0, L)], buf, sem).wait()

    def acc_row(i, buf):
        def acc_body(j4, accs):
            j = j4 * 4
            for u in range(4):
                accs = tuple(accs[k] + buf[j + u, pl.ds(16 * k, 16)]
                             for k in range(4))
            return accs

        accs = (jnp.zeros((16,), jnp.float32),) * 4
        accs = lax.fori_loop(0, L // 4, acc_body, accs)
        for k in range(4):
            out_v[i, pl.ds(16 * k, 16)] = accs[k]

    issue(0, rows0, sem0)

    def pair_body(i2, carry):
        b0 = 2 * i2
        b1 = b0 + 1
        issue(b1, rows1, sem1)
        drain(rows0, sem0)
        acc_row(b0, rows0)
        issue(jnp.minimum(b0 + 2, BPW - 1), rows0, sem0)
        drain(rows1, sem1)
        acc_row(b1, rows1)
        return carry

    lax.fori_loop(0, BPW // 2, pair_body, 0)
    drain(rows0, sem0)
    pltpu.sync_copy(out_v, out_hbm.at[pl.ds(base, BPW)])


def _make_pool():
    mesh = plsc.VectorSubcoreMesh(core_axis_name="c", subcore_axis_name="s")
    return pl.kernel(
        _pool_body,
        out_type=jax.ShapeDtypeStruct((B, EMB), jnp.float32),
        mesh=mesh,
        scratch_types=[
            pltpu.VMEM((BPW, L), jnp.int32),
            pltpu.VMEM((L, EMB), jnp.float32),
            pltpu.VMEM((L, EMB), jnp.float32),
            pltpu.VMEM((BPW, EMB), jnp.float32),
            pltpu.SemaphoreType.DMA,
            pltpu.SemaphoreType.DMA,
        ],
        compiler_params=pltpu.CompilerParams(use_tc_tiling_on_sc=False),
    )


PACK_BN = 2048


def _pack_body(t_ref, o_ref):
    x = t_ref[...]
    eye = (lax.broadcasted_iota(jnp.int32, (EMB, EMB), 0)
           == lax.broadcasted_iota(jnp.int32, (EMB, EMB), 1)
           ).astype(jnp.float32)
    xt = lax.dot_general(x, eye, (((0,), (0,)), ((), ())),
                         preferred_element_type=jnp.float32)
    t3 = xt.reshape(PACK_BN // 2, 2, EMB)
    o_ref[...] = jnp.concatenate([t3[:, 0, :], t3[:, 1, :]], axis=1)


def _make_pack():
    grid = (VOCAB + PACK_BN - 1) // PACK_BN
    return pl.pallas_call(
        _pack_body,
        grid=(grid,),
        in_specs=[pl.BlockSpec((EMB, PACK_BN), lambda i: (0, i))],
        out_specs=pl.BlockSpec((PACK_BN // 2, 2 * EMB), lambda i: (i, 0)),
        out_shape=jax.ShapeDtypeStruct((VOCAB // 2, 2 * EMB), jnp.float32),
    )


def _mm_body(x_ref, w_ref, b_ref, o_ref):
    x = x_ref[...] * jnp.float32(1.0 / L)
    o_ref[...] = (
        lax.dot_general(w_ref[...], x, (((0,), (1,)), ((), ())),
                        preferred_element_type=jnp.float32)
        + b_ref[...]
    )


def _make_mm(bm):
    return pl.pallas_call(
        _mm_body,
        grid=(B // bm,),
        in_specs=[
            pl.BlockSpec((bm, EMB), lambda i: (i, 0)),
            pl.BlockSpec((EMB, NUM_CLASS), lambda i: (0, 0)),
            pl.BlockSpec((NUM_CLASS, 1), lambda i: (0, 0)),
        ],
        out_specs=pl.BlockSpec((NUM_CLASS, bm), lambda i: (0, i)),
        out_shape=jax.ShapeDtypeStruct((NUM_CLASS, B), jnp.float32),
    )


def kernel(input, table, W, b):
    idx = input.astype(jnp.int32)
    t2 = _make_pack()(table.T)
    t_lin = jnp.reshape(t2, (VOCAB, EMB))
    pooled = _make_pool()(idx, t_lin)
    logits_t = _make_mm(512)(pooled, W, b.reshape(NUM_CLASS, 1))
    return logits_t.T


# speedup vs baseline: 1.1541x; 1.1220x over previous
import functools

import jax
import jax.numpy as jnp
from jax import lax
from jax.experimental import pallas as pl
from jax.experimental.pallas import tpu as pltpu
from jax.experimental.pallas import tpu_sc as plsc

VOCAB = 1000000
EMB = 64
NUM_CLASS = 1000
B = 4096
L = 200

NC = 2
NS = 16
NW = NC * NS
BPW = B // NW
CHUNKS = ((0, 104), (104, 96))


def _pool_body(idx_hbm, table_hbm, out_hbm, idx_v, rows0, rows1, out_v,
               sem0, sem1):
    c = lax.axis_index("c")
    s = lax.axis_index("s")
    wid = s * NC + c
    base = wid * BPW

    pltpu.sync_copy(idx_hbm.at[pl.ds(base, BPW)], idx_v)

    def issue(i, buf, sem):
        for off, sz in CHUNKS:
            pltpu.async_copy(
                table_hbm.at[idx_v.at[i, pl.ds(off, sz)]],
                buf.at[pl.ds(off, sz)],
                sem,
            )

    def drain(buf, sem):
        pltpu.make_async_copy(table_hbm.at[pl.ds(0, L)], buf, sem).wait()

    def acc_row(i, buf):
        def acc_body(j4, accs):
            j = j4 * 4
            for u in range(4):
                accs = tuple(accs[k] + buf[j + u, pl.ds(16 * k, 16)]
                             for k in range(4))
            return accs

        accs = (jnp.zeros((16,), jnp.float32),) * 4
        accs = lax.fori_loop(0, L // 4, acc_body, accs)
        for k in range(4):
            out_v[i, pl.ds(16 * k, 16)] = accs[k]

    issue(0, rows0, sem0)

    def pair_body(i2, carry):
        b0 = 2 * i2
        b1 = b0 + 1
        issue(b1, rows1, sem1)
        drain(rows0, sem0)
        acc_row(b0, rows0)
        issue(jnp.minimum(b0 + 2, BPW - 1), rows0, sem0)
        drain(rows1, sem1)
        acc_row(b1, rows1)
        return carry

    lax.fori_loop(0, BPW // 2, pair_body, 0)
    drain(rows0, sem0)
    pltpu.sync_copy(out_v, out_hbm.at[pl.ds(base, BPW)])


def _make_pool():
    mesh = plsc.VectorSubcoreMesh(core_axis_name="c", subcore_axis_name="s")
    return pl.kernel(
        _pool_body,
        out_type=jax.ShapeDtypeStruct((B, EMB), jnp.float32),
        mesh=mesh,
        scratch_types=[
            pltpu.VMEM((BPW, L), jnp.int32),
            pltpu.VMEM((L, EMB), jnp.float32),
            pltpu.VMEM((L, EMB), jnp.float32),
            pltpu.VMEM((BPW, EMB), jnp.float32),
            pltpu.SemaphoreType.DMA,
            pltpu.SemaphoreType.DMA,
        ],
        compiler_params=pltpu.CompilerParams(use_tc_tiling_on_sc=False),
    )


PACK_BN = 2048


def _pack_body(t_ref, o_ref):
    x = t_ref[...]
    eye = (lax.broadcasted_iota(jnp.int32, (EMB, EMB), 0)
           == lax.broadcasted_iota(jnp.int32, (EMB, EMB), 1)
           ).astype(jnp.float32)
    xt = lax.dot_general(x, eye, (((0,), (0,)), ((), ())),
                         preferred_element_type=jnp.float32)
    o_ref[:, :EMB] = xt


def _make_pack():
    grid = (VOCAB + PACK_BN - 1) // PACK_BN
    return pl.pallas_call(
        _pack_body,
        grid=(grid,),
        in_specs=[pl.BlockSpec((EMB, PACK_BN), lambda i: (0, i))],
        out_specs=pl.BlockSpec((PACK_BN, 128), lambda i: (i, 0)),
        out_shape=jax.ShapeDtypeStruct((VOCAB, 128), jnp.float32),
    )


def _mm_body(x_ref, w_ref, b_ref, o_ref):
    x = x_ref[...] * jnp.float32(1.0 / L)
    o_ref[...] = (
        lax.dot_general(w_ref[...], x, (((0,), (1,)), ((), ())),
                        preferred_element_type=jnp.float32)
        + b_ref[...]
    )


def _make_mm(bm):
    return pl.pallas_call(
        _mm_body,
        grid=(B // bm,),
        in_specs=[
            pl.BlockSpec((bm, EMB), lambda i: (i, 0)),
            pl.BlockSpec((EMB, NUM_CLASS), lambda i: (0, 0)),
            pl.BlockSpec((NUM_CLASS, 1), lambda i: (0, 0)),
        ],
        out_specs=pl.BlockSpec((NUM_CLASS, bm), lambda i: (0, i)),
        out_shape=jax.ShapeDtypeStruct((NUM_CLASS, B), jnp.float32),
    )


def kernel(input, table, W, b):
    idx = input.astype(jnp.int32) * 2
    t2 = _make_pack()(table.T)
    t_lin = jnp.reshape(t2, (2 * VOCAB, EMB))
    pooled = _make_pool()(idx, t_lin)
    logits_t = _make_mm(512)(pooled, W, b.reshape(NUM_CLASS, 1))
    return logits_t.T


# speedup vs baseline: 1.4741x; 1.2772x over previous
import functools

import jax
import jax.numpy as jnp
from jax import lax
from jax.experimental import pallas as pl
from jax.experimental.pallas import tpu as pltpu
from jax.experimental.pallas import tpu_sc as plsc

VOCAB = 1000000
EMB = 64
NUM_CLASS = 1000
B = 4096
L = 200

NC = 2
NS = 16
NW = NC * NS
BPW = B // NW
CHUNKS = ((0, 104), (104, 96))


def _pool_body(idx_hbm, table_hbm, out_hbm, idx_v, rows0, rows1, out_v,
               sem0, sem1):
    c = lax.axis_index("c")
    s = lax.axis_index("s")
    wid = s * NC + c
    base = wid * BPW

    pltpu.sync_copy(idx_hbm.at[pl.ds(base, BPW)], idx_v)

    def issue(i, buf, sem):
        for off, sz in CHUNKS:
            pltpu.async_copy(
                table_hbm.at[idx_v.at[i, pl.ds(off, sz)]],
                buf.at[pl.ds(off, sz)],
                sem,
            )

    def drain(buf, sem):
        pltpu.make_async_copy(table_hbm.at[pl.ds(0, L)], buf, sem).wait()

    def acc_row(i, buf):
        def acc_body(j4, accs):
            j = j4 * 4
            for u in range(4):
                accs = tuple(accs[k] + buf[j + u, pl.ds(16 * k, 16)]
                             for k in range(4))
            return accs

        accs = (jnp.zeros((16,), jnp.float32),) * 4
        accs = lax.fori_loop(0, L // 4, acc_body, accs)
        for k in range(4):
            out_v[i, pl.ds(16 * k, 16)] = accs[k]

    issue(0, rows0, sem0)

    def pair_body(i2, carry):
        b0 = 2 * i2
        b1 = b0 + 1
        issue(b1, rows1, sem1)
        drain(rows0, sem0)
        acc_row(b0, rows0)
        issue(jnp.minimum(b0 + 2, BPW - 1), rows0, sem0)
        drain(rows1, sem1)
        acc_row(b1, rows1)
        return carry

    lax.fori_loop(0, BPW // 2, pair_body, 0)
    drain(rows0, sem0)
    pltpu.sync_copy(out_v, out_hbm.at[pl.ds(base, BPW)])


def _make_pool():
    mesh = plsc.VectorSubcoreMesh(core_axis_name="c", subcore_axis_name="s")
    return pl.kernel(
        _pool_body,
        out_type=jax.ShapeDtypeStruct((B, EMB), jnp.float32),
        mesh=mesh,
        scratch_types=[
            pltpu.VMEM((BPW, L), jnp.int32),
            pltpu.VMEM((L, EMB), jnp.float32),
            pltpu.VMEM((L, EMB), jnp.float32),
            pltpu.VMEM((BPW, EMB), jnp.float32),
            pltpu.SemaphoreType.DMA,
            pltpu.SemaphoreType.DMA,
        ],
        compiler_params=pltpu.CompilerParams(use_tc_tiling_on_sc=False),
    )


PACK_BN = 4096


def _pack_body(t_ref, o_ref):
    x = t_ref[...]
    eye = (lax.broadcasted_iota(jnp.int32, (EMB, EMB), 0)
           == lax.broadcasted_iota(jnp.int32, (EMB, EMB), 1)
           ).astype(jnp.float32)
    xt = lax.dot_general(x, eye, (((0,), (0,)), ((), ())),
                         preferred_element_type=jnp.float32)
    o_ref[:, :EMB] = xt


def _make_pack():
    grid = (VOCAB + PACK_BN - 1) // PACK_BN
    return pl.pallas_call(
        _pack_body,
        grid=(grid,),
        in_specs=[pl.BlockSpec((EMB, PACK_BN), lambda i: (0, i))],
        out_specs=pl.BlockSpec((PACK_BN, 128), lambda i: (i, 0)),
        out_shape=jax.ShapeDtypeStruct((VOCAB, 128), jnp.float32),
    )


def _mm_body(x_ref, w_ref, b_ref, o_ref):
    x = x_ref[...] * jnp.float32(1.0 / L)
    o_ref[...] = (
        lax.dot_general(w_ref[...], x, (((0,), (1,)), ((), ())),
                        preferred_element_type=jnp.float32)
        + b_ref[...]
    )


def _make_mm(bm):
    return pl.pallas_call(
        _mm_body,
        grid=(B // bm,),
        in_specs=[
            pl.BlockSpec((bm, EMB), lambda i: (i, 0)),
            pl.BlockSpec((EMB, NUM_CLASS), lambda i: (0, 0)),
            pl.BlockSpec((NUM_CLASS, 1), lambda i: (0, 0)),
        ],
        out_specs=pl.BlockSpec((NUM_CLASS, bm), lambda i: (0, i)),
        out_shape=jax.ShapeDtypeStruct((NUM_CLASS, B), jnp.float32),
    )


def kernel(input, table, W, b):
    idx = input.astype(jnp.int32) * 2
    t2 = _make_pack()(table.T)
    t_lin = jnp.reshape(t2, (2 * VOCAB, EMB))
    pooled = _make_pool()(idx, t_lin)
    logits_t = _make_mm(512)(pooled, W, b.reshape(NUM_CLASS, 1))
    return logits_t.T


# speedup vs baseline: 1.7414x; 1.1813x over previous
import functools

import jax
import jax.numpy as jnp
from jax import lax
from jax.experimental import pallas as pl
from jax.experimental.pallas import tpu as pltpu
from jax.experimental.pallas import tpu_sc as plsc

VOCAB = 1000000
EMB = 64
NUM_CLASS = 1000
B = 4096
L = 200

NC = 2
NS = 16
NW = NC * NS
BPW = B // NW
CHUNKS = ((0, 104), (104, 96))


def _pool_body(idx_hbm, table_hbm, out_hbm, idx_v, rows0, rows1, out_v,
               sem0, sem1):
    c = lax.axis_index("c")
    s = lax.axis_index("s")
    wid = s * NC + c
    base = wid * BPW

    pltpu.sync_copy(idx_hbm.at[pl.ds(base, BPW)], idx_v)

    def issue(i, buf, sem):
        for off, sz in CHUNKS:
            pltpu.async_copy(
                table_hbm.at[idx_v.at[i, pl.ds(off, sz)]],
                buf.at[pl.ds(off, sz)],
                sem,
            )

    def drain(buf, sem):
        pltpu.make_async_copy(table_hbm.at[pl.ds(0, L)], buf, sem).wait()

    def acc_row(i, buf):
        def acc_body(j4, accs):
            j = j4 * 4
            for u in range(4):
                accs = tuple(accs[k] + buf[j + u, pl.ds(16 * k, 16)]
                             for k in range(4))
            return accs

        accs = (jnp.zeros((16,), jnp.float32),) * 4
        accs = lax.fori_loop(0, L // 4, acc_body, accs)
        for k in range(4):
            out_v[i, pl.ds(16 * k, 16)] = accs[k]

    issue(0, rows0, sem0)

    def pair_body(i2, carry):
        b0 = 2 * i2
        b1 = b0 + 1
        issue(b1, rows1, sem1)
        drain(rows0, sem0)
        acc_row(b0, rows0)
        issue(jnp.minimum(b0 + 2, BPW - 1), rows0, sem0)
        drain(rows1, sem1)
        acc_row(b1, rows1)
        return carry

    lax.fori_loop(0, BPW // 2, pair_body, 0)
    drain(rows0, sem0)
    pltpu.sync_copy(out_v, out_hbm.at[pl.ds(base, BPW)])


def _make_pool():
    mesh = plsc.VectorSubcoreMesh(core_axis_name="c", subcore_axis_name="s")
    return pl.kernel(
        _pool_body,
        out_type=jax.ShapeDtypeStruct((B, EMB), jnp.float32),
        mesh=mesh,
        scratch_types=[
            pltpu.VMEM((BPW, L), jnp.int32),
            pltpu.VMEM((L, EMB), jnp.float32),
            pltpu.VMEM((L, EMB), jnp.float32),
            pltpu.VMEM((BPW, EMB), jnp.float32),
            pltpu.SemaphoreType.DMA,
            pltpu.SemaphoreType.DMA,
        ],
        compiler_params=pltpu.CompilerParams(use_tc_tiling_on_sc=False),
    )


PACK_BN = 8192


def _pack_body(t_ref, o_ref):
    x = t_ref[...]
    eye = (lax.broadcasted_iota(jnp.int32, (EMB, EMB), 0)
           == lax.broadcasted_iota(jnp.int32, (EMB, EMB), 1)
           ).astype(jnp.float32)
    xt = lax.dot_general(x, eye, (((0,), (0,)), ((), ())),
                         preferred_element_type=jnp.float32)
    o_ref[:, :EMB] = xt


def _make_pack():
    grid = (VOCAB + PACK_BN - 1) // PACK_BN
    return pl.pallas_call(
        _pack_body,
        grid=(grid,),
        in_specs=[pl.BlockSpec((EMB, PACK_BN), lambda i: (0, i))],
        out_specs=pl.BlockSpec((PACK_BN, 128), lambda i: (i, 0)),
        out_shape=jax.ShapeDtypeStruct((VOCAB, 128), jnp.float32),
    )


def _mm_body(x_ref, w_ref, b_ref, o_ref):
    x = x_ref[...] * jnp.float32(1.0 / L)
    o_ref[...] = (
        lax.dot_general(w_ref[...], x, (((0,), (1,)), ((), ())),
                        preferred_element_type=jnp.float32)
        + b_ref[...]
    )


def _make_mm(bm):
    return pl.pallas_call(
        _mm_body,
        grid=(B // bm,),
        in_specs=[
            pl.BlockSpec((bm, EMB), lambda i: (i, 0)),
            pl.BlockSpec((EMB, NUM_CLASS), lambda i: (0, 0)),
            pl.BlockSpec((NUM_CLASS, 1), lambda i: (0, 0)),
        ],
        out_specs=pl.BlockSpec((NUM_CLASS, bm), lambda i: (0, i)),
        out_shape=jax.ShapeDtypeStruct((NUM_CLASS, B), jnp.float32),
    )


def kernel(input, table, W, b):
    idx = input.astype(jnp.int32) * 2
    t2 = _make_pack()(table.T)
    t_lin = jnp.reshape(t2, (2 * VOCAB, EMB))
    pooled = _make_pool()(idx, t_lin)
    logits_t = _make_mm(512)(pooled, W, b.reshape(NUM_CLASS, 1))
    return logits_t.T


# speedup vs baseline: 1.8440x; 1.0589x over previous
import functools

import jax
import jax.numpy as jnp
from jax import lax
from jax.experimental import pallas as pl
from jax.experimental.pallas import tpu as pltpu
from jax.experimental.pallas import tpu_sc as plsc

VOCAB = 1000000
EMB = 64
NUM_CLASS = 1000
B = 4096
L = 200

NC = 2
NS = 16
NW = NC * NS
BPW = B // NW
CHUNKS = ((0, 104), (104, 96))


def _pool_body(idx_hbm, table_hbm, out_hbm, idx_v, rows0, rows1, out_v,
               sem0, sem1):
    c = lax.axis_index("c")
    s = lax.axis_index("s")
    wid = s * NC + c
    base = wid * BPW

    pltpu.sync_copy(idx_hbm.at[pl.ds(base, BPW)], idx_v)

    def issue(i, buf, sem):
        for off, sz in CHUNKS:
            pltpu.async_copy(
                table_hbm.at[idx_v.at[i, pl.ds(off, sz)]],
                buf.at[pl.ds(off, sz)],
                sem,
            )

    def drain(buf, sem):
        pltpu.make_async_copy(table_hbm.at[pl.ds(0, L)], buf, sem).wait()

    def acc_row(i, buf):
        def acc_body(j4, accs):
            j = j4 * 4
            for u in range(4):
                accs = tuple(accs[k] + buf[j + u, pl.ds(16 * k, 16)]
                             for k in range(4))
            return accs

        accs = (jnp.zeros((16,), jnp.float32),) * 4
        accs = lax.fori_loop(0, L // 4, acc_body, accs)
        for k in range(4):
            out_v[i, pl.ds(16 * k, 16)] = accs[k]

    issue(0, rows0, sem0)

    def pair_body(i2, carry):
        b0 = 2 * i2
        b1 = b0 + 1
        issue(b1, rows1, sem1)
        drain(rows0, sem0)
        acc_row(b0, rows0)
        issue(jnp.minimum(b0 + 2, BPW - 1), rows0, sem0)
        drain(rows1, sem1)
        acc_row(b1, rows1)
        return carry

    lax.fori_loop(0, BPW // 2, pair_body, 0)
    drain(rows0, sem0)
    pltpu.sync_copy(out_v, out_hbm.at[pl.ds(base, BPW)])


def _make_pool():
    mesh = plsc.VectorSubcoreMesh(core_axis_name="c", subcore_axis_name="s")
    return pl.kernel(
        _pool_body,
        out_type=jax.ShapeDtypeStruct((B, EMB), jnp.float32),
        mesh=mesh,
        scratch_types=[
            pltpu.VMEM((BPW, L), jnp.int32),
            pltpu.VMEM((L, EMB), jnp.float32),
            pltpu.VMEM((L, EMB), jnp.float32),
            pltpu.VMEM((BPW, EMB), jnp.float32),
            pltpu.SemaphoreType.DMA,
            pltpu.SemaphoreType.DMA,
        ],
        compiler_params=pltpu.CompilerParams(use_tc_tiling_on_sc=False),
    )


PACK_BN = 16384


def _pack_body(t_ref, o_ref):
    x = t_ref[...]
    eye = (lax.broadcasted_iota(jnp.int32, (EMB, EMB), 0)
           == lax.broadcasted_iota(jnp.int32, (EMB, EMB), 1)
           ).astype(jnp.float32)
    xt = lax.dot_general(x, eye, (((0,), (0,)), ((), ())),
                         preferred_element_type=jnp.float32)
    o_ref[:, :EMB] = xt


def _make_pack():
    grid = (VOCAB + PACK_BN - 1) // PACK_BN
    return pl.pallas_call(
        _pack_body,
        grid=(grid,),
        in_specs=[pl.BlockSpec((EMB, PACK_BN), lambda i: (0, i))],
        out_specs=pl.BlockSpec((PACK_BN, 128), lambda i: (i, 0)),
        out_shape=jax.ShapeDtypeStruct((VOCAB, 128), jnp.float32),
    )


def _mm_body(x_ref, w_ref, b_ref, o_ref):
    x = x_ref[...] * jnp.float32(1.0 / L)
    o_ref[...] = (
        lax.dot_general(w_ref[...], x, (((0,), (1,)), ((), ())),
                        preferred_element_type=jnp.float32)
        + b_ref[...]
    )


def _make_mm(bm):
    return pl.pallas_call(
        _mm_body,
        grid=(B // bm,),
        in_specs=[
            pl.BlockSpec((bm, EMB), lambda i: (i, 0)),
            pl.BlockSpec((EMB, NUM_CLASS), lambda i: (0, 0)),
            pl.BlockSpec((NUM_CLASS, 1), lambda i: (0, 0)),
        ],
        out_specs=pl.BlockSpec((NUM_CLASS, bm), lambda i: (0, i)),
        out_shape=jax.ShapeDtypeStruct((NUM_CLASS, B), jnp.float32),
    )


def kernel(input, table, W, b):
    idx = input.astype(jnp.int32) * 2
    t2 = _make_pack()(table.T)
    t_lin = jnp.reshape(t2, (2 * VOCAB, EMB))
    pooled = _make_pool()(idx, t_lin)
    logits_t = _make_mm(512)(pooled, W, b.reshape(NUM_CLASS, 1))
    return logits_t.T
